# Initial kernel scaffold; baseline (speedup 1.0000x reference)
import jax
import jax.numpy as jnp
from jax.experimental import pallas as pl


def kernel(inputs, B_param, C_param, D, enc_w, enc_b, conv_w, conv_b, r_logit_base, th_atanh_base, r_head_w, th_head_w, dt_base, inj_head_w):
    raise NotImplementedError("write your pallas kernel here")


# speedup vs baseline: 52.7815x; 52.7815x over previous
import functools
import math

import jax
import jax.numpy as jnp
from jax.experimental import pallas as pl
from jax.experimental.pallas import tpu as pltpu

H = 256
M = 256
K = 4

_COS_COEF = (
    0.9999999998846172,
    -0.49999999850320187,
    0.04166666347767178,
    -0.0013888862973696803,
    2.4800551328531826e-05,
    -2.753476742453543e-07,
    2.0603329868034754e-09,
    -9.721733557966761e-12,
)


def _cos_poly(theta):
    u = theta * theta
    acc = jnp.full_like(u, _COS_COEF[-1])
    for c in _COS_COEF[-2::-1]:
        acc = acc * u + c
    return acc


def _silu(x):
    return x * jax.nn.sigmoid(x)


def _dlinoss_kernel(
    x_ref,
    enc_wT_ref,
    enc_b_ref,
    convw_ref,
    conv_b_ref,
    rw_T_ref,
    tw_T_ref,
    iw_T_ref,
    b0T_ref,
    b1T_ref,
    c0T_ref,
    c1T_ref,
    rb_ref,
    tb_ref,
    dtb_ref,
    d_ref,
    o_ref,
    tail_ref,
    zr_ref, zi_ref, xr_ref, xi_ref,
    sv_ref, p_ref, bzr_ref, bzi_ref,
    xsr_ref, xsi_ref,
    *, T):
    c = pl.program_id(1)
    T8 = T * 8

    @pl.when(c == 0)
    def _init():
        tail_ref[...] = jnp.zeros_like(tail_ref)
        zr_ref[...] = jnp.zeros_like(zr_ref)
        zi_ref[...] = jnp.zeros_like(zi_ref)
        xr_ref[...] = jnp.zeros_like(xr_ref)
        xi_ref[...] = jnp.zeros_like(xi_ref)

    x3 = x_ref[...]
    x2 = x3.reshape(T8, H)

    pre2 = _silu(
        jax.lax.dot_general(x2, enc_wT_ref[...],
                            (((1,), (0,)), ((), ())),
                            preferred_element_type=jnp.float32)
        + enc_b_ref[...])
    pre3 = pre2.reshape(T, 8, H)

    full = jnp.concatenate([tail_ref[...], pre3], axis=0)
    tail_ref[...] = pre3[T - (K - 1):]
    w = convw_ref[...]
    conv3 = (w[0].reshape(1, 1, H) * full[0:T]
             + w[1].reshape(1, 1, H) * full[1:T + 1]
             + w[2].reshape(1, 1, H) * full[2:T + 2]
             + w[3].reshape(1, 1, H) * pre3)
    feats2 = _silu(conv3.reshape(T8, H) + conv_b_ref[...])

    r = jax.nn.sigmoid(
        rb_ref[...] + jax.lax.dot_general(
            feats2, rw_T_ref[...], (((1,), (0,)), ((), ())),
            preferred_element_type=jnp.float32))
    theta = math.pi * jnp.tanh(
        tb_ref[...] + jax.lax.dot_general(
            feats2, tw_T_ref[...], (((1,), (0,)), ((), ())),
            preferred_element_type=jnp.float32))
    gate = jax.nn.sigmoid(
        jax.lax.dot_general(feats2, iw_T_ref[...], (((1,), (0,)), ((), ())),
                            preferred_element_type=jnp.float32))

    dtc = jnp.maximum(jax.nn.sigmoid(dtb_ref[...]), 1e-6)
    inv_dtc = 1.0 / dtc
    r2 = jnp.maximum(r * r, 1e-8)
    inv_r2 = 1.0 / r2
    A = jnp.maximum((r2 - 2.0 * r * _cos_poly(theta) + 1.0)
                    * inv_r2 * (inv_dtc * inv_dtc), 0.0)
    G = jnp.maximum((1.0 - r2) * inv_r2 * inv_dtc, 0.0)
    sinv = 1.0 / jnp.maximum(1.0 + dtc * G, 1e-6)
    ds = dtc * sinv

    bur = jax.lax.dot_general(x2, b0T_ref[...], (((1,), (0,)), ((), ())),
                              preferred_element_type=jnp.float32) * gate
    bui = jax.lax.dot_general(x2, b1T_ref[...], (((1,), (0,)), ((), ())),
                              preferred_element_type=jnp.float32) * gate

    sv_ref[...] = sinv
    p_ref[...] = -ds * A
    bzr_ref[...] = ds * bur
    bzi_ref[...] = ds * bui

    def step(t, carry):
        zr, zi, xr, xi = carry
        b = pl.multiple_of(t * 8, 8)
        sv = sv_ref[pl.ds(b, 8), :]
        pv = p_ref[pl.ds(b, 8), :]
        zr = sv * zr + pv * xr + bzr_ref[pl.ds(b, 8), :]
        zi = sv * zi + pv * xi + bzi_ref[pl.ds(b, 8), :]
        xr = xr + dtc * zr
        xi = xi + dtc * zi
        xsr_ref[pl.ds(b, 8), :] = xr
        xsi_ref[pl.ds(b, 8), :] = xi
        return (zr, zi, xr, xi)

    init = (zr_ref[...], zi_ref[...], xr_ref[...], xi_ref[...])
    zr, zi, xr, xi = jax.lax.fori_loop(0, T, step, init)
    zr_ref[...] = zr
    zi_ref[...] = zi
    xr_ref[...] = xr
    xi_ref[...] = xi

    proj = (jax.lax.dot_general(xsr_ref[...], c0T_ref[...],
                                (((1,), (0,)), ((), ())),
                                preferred_element_type=jnp.float32)
            - jax.lax.dot_general(xsi_ref[...], c1T_ref[...],
                                  (((1,), (0,)), ((), ())),
                                  preferred_element_type=jnp.float32))
    o_ref[...] = proj.reshape(T, 8, H) + d_ref[...].reshape(1, 1, H) * x3


def kernel(inputs, B_param, C_param, D, enc_w, enc_b, conv_w, conv_b,
           r_logit_base, th_atanh_base, r_head_w, th_head_w, dt_base,
           inj_head_w):
    B, L, _ = inputs.shape
    T = 128
    NB = B // 8
    NC = L // T

    x_t = jnp.transpose(inputs, (1, 0, 2))
    enc_wT = enc_w.T
    convw = conv_w[:, 0, :].T
    rw_T = r_head_w.T
    tw_T = th_head_w.T
    iw_T = inj_head_w.T
    b0T = B_param[..., 0].T
    b1T = B_param[..., 1].T
    c0T = C_param[..., 0].T
    c1T = C_param[..., 1].T
    rb = r_logit_base.reshape(1, M)
    tb = th_atanh_base.reshape(1, M)
    dtb = dt_base.reshape(1, M)
    enc_b2 = enc_b.reshape(1, H)
    conv_b2 = conv_b.reshape(1, H)
    d2 = D.reshape(1, H)

    def fixed(shape):
        return pl.BlockSpec(shape, lambda b, c: tuple(0 for _ in shape))

    out = pl.pallas_call(
        functools.partial(_dlinoss_kernel, T=T),
        out_shape=jax.ShapeDtypeStruct((L, B, H), jnp.float32),
        grid=(NB, NC),
        in_specs=[
            pl.BlockSpec((T, 8, H), lambda b, c: (c, b, 0)),
            fixed((H, H)), fixed((1, H)), fixed((K, H)), fixed((1, H)),
            fixed((H, M)), fixed((H, M)), fixed((H, M)),
            fixed((H, M)), fixed((H, M)),
            fixed((M, H)), fixed((M, H)),
            fixed((1, M)), fixed((1, M)), fixed((1, M)), fixed((1, H)),
        ],
        out_specs=pl.BlockSpec((T, 8, H), lambda b, c: (c, b, 0)),
        scratch_shapes=[
            pltpu.VMEM((K - 1, 8, H), jnp.float32),
            pltpu.VMEM((8, M), jnp.float32),
            pltpu.VMEM((8, M), jnp.float32),
            pltpu.VMEM((8, M), jnp.float32),
            pltpu.VMEM((8, M), jnp.float32),
            pltpu.VMEM((T * 8, M), jnp.float32),
            pltpu.VMEM((T * 8, M), jnp.float32),
            pltpu.VMEM((T * 8, M), jnp.float32),
            pltpu.VMEM((T * 8, M), jnp.float32),
            pltpu.VMEM((T * 8, M), jnp.float32),
            pltpu.VMEM((T * 8, M), jnp.float32),
        ],
        compiler_params=pltpu.CompilerParams(
            dimension_semantics=("parallel", "arbitrary"),
            vmem_limit_bytes=100 * 1024 * 1024,
        ),
        name="selective_dlinoss",
    )(x_t, enc_wT, enc_b2, convw, conv_b2, rw_T, tw_T, iw_T,
      b0T, b1T, c0T, c1T, rb, tb, dtb, d2)

    return jnp.transpose(out, (1, 0, 2))


# speedup vs baseline: 58.7205x; 1.1125x over previous
import functools
import math

import jax
import jax.numpy as jnp
from jax.experimental import pallas as pl
from jax.experimental.pallas import tpu as pltpu

H = 256
M = 256
K = 4
NB = 16

_COS_COEF = (
    0.9999999998846172,
    -0.49999999850320187,
    0.04166666347767178,
    -0.0013888862973696803,
    2.4800551328531826e-05,
    -2.753476742453543e-07,
    2.0603329868034754e-09,
    -9.721733557966761e-12,
)


def _cos_poly(theta):
    u = theta * theta
    acc = jnp.full_like(u, _COS_COEF[-1])
    for c in _COS_COEF[-2::-1]:
        acc = acc * u + c
    return acc


def _silu(x):
    return x * jax.nn.sigmoid(x)


def _mm(x, w):
    return jax.lax.dot_general(x, w, (((1,), (0,)), ((), ())),
                               preferred_element_type=jnp.float32)


def _dlinoss_kernel(
    x_ref,
    enc_wT_ref,
    enc_b_ref,
    convw_ref,
    conv_b_ref,
    rw_T_ref,
    tw_T_ref,
    iw_T_ref,
    b0T_ref,
    b1T_ref,
    c0T_ref,
    c1T_ref,
    rb_ref,
    tb_ref,
    dtb_ref,
    d_ref,
    o_ref,
    tail_ref,
    zr_ref, zi_ref, xr_ref, xi_ref,
    sv_ref, p_ref, bzr_ref, bzi_ref,
    xsr_ref, xsi_ref,
    *, T):
    c = pl.program_id(0)
    TN = T * NB

    @pl.when(c == 0)
    def _init():
        tail_ref[...] = jnp.zeros_like(tail_ref)
        zr_ref[...] = jnp.zeros_like(zr_ref)
        zi_ref[...] = jnp.zeros_like(zi_ref)
        xr_ref[...] = jnp.zeros_like(xr_ref)
        xi_ref[...] = jnp.zeros_like(xi_ref)

    x3 = x_ref[...]
    x2 = x3.reshape(TN, H)

    pre2 = _silu(_mm(x2, enc_wT_ref[...]) + enc_b_ref[...])
    pre3 = pre2.reshape(T, NB, H)

    full = jnp.concatenate([tail_ref[...], pre3], axis=0)
    tail_ref[...] = pre3[T - (K - 1):]
    w = convw_ref[...]
    conv3 = (w[0].reshape(1, 1, H) * full[0:T]
             + w[1].reshape(1, 1, H) * full[1:T + 1]
             + w[2].reshape(1, 1, H) * full[2:T + 2]
             + w[3].reshape(1, 1, H) * pre3)
    feats2 = _silu(conv3.reshape(TN, H) + conv_b_ref[...])

    r = jax.nn.sigmoid(rb_ref[...] + _mm(feats2, rw_T_ref[...]))
    theta = math.pi * jnp.tanh(tb_ref[...] + _mm(feats2, tw_T_ref[...]))
    gate = jax.nn.sigmoid(_mm(feats2, iw_T_ref[...]))

    dtc = jnp.maximum(jax.nn.sigmoid(dtb_ref[...]), 1e-6)
    inv_dtc = 1.0 / dtc
    r2 = jnp.maximum(r * r, 1e-8)
    inv_r2 = 1.0 / r2
    A = jnp.maximum((r2 - 2.0 * r * _cos_poly(theta) + 1.0)
                    * inv_r2 * (inv_dtc * inv_dtc), 0.0)
    G = jnp.maximum((1.0 - r2) * inv_r2 * inv_dtc, 0.0)
    sinv = 1.0 / jnp.maximum(1.0 + dtc * G, 1e-6)
    ds = dtc * sinv

    bur = _mm(x2, b0T_ref[...]) * gate
    bui = _mm(x2, b1T_ref[...]) * gate

    sv_ref[...] = sinv
    p_ref[...] = -ds * A
    bzr_ref[...] = ds * bur
    bzi_ref[...] = ds * bui

    def step(t, carry):
        zr, zi, xr, xi = carry
        b = pl.multiple_of(t * NB, NB)
        sv = sv_ref[pl.ds(b, NB), :]
        pv = p_ref[pl.ds(b, NB), :]
        zr = sv * zr + pv * xr + bzr_ref[pl.ds(b, NB), :]
        zi = sv * zi + pv * xi + bzi_ref[pl.ds(b, NB), :]
        xr = xr + dtc * zr
        xi = xi + dtc * zi
        xsr_ref[pl.ds(b, NB), :] = xr
        xsi_ref[pl.ds(b, NB), :] = xi
        return (zr, zi, xr, xi)

    init = (zr_ref[...], zi_ref[...], xr_ref[...], xi_ref[...])
    zr, zi, xr, xi = jax.lax.fori_loop(0, T, step, init)
    zr_ref[...] = zr
    zi_ref[...] = zi
    xr_ref[...] = xr
    xi_ref[...] = xi

    proj = _mm(xsr_ref[...], c0T_ref[...]) - _mm(xsi_ref[...], c1T_ref[...])
    o_ref[...] = proj.reshape(T, NB, H) + d_ref[...].reshape(1, 1, H) * x3


def kernel(inputs, B_param, C_param, D, enc_w, enc_b, conv_w, conv_b,
           r_logit_base, th_atanh_base, r_head_w, th_head_w, dt_base,
           inj_head_w):
    B, L, _ = inputs.shape
    T = 128
    NC = L // T

    x_t = jnp.transpose(inputs, (1, 0, 2))
    enc_wT = enc_w.T
    convw = conv_w[:, 0, :].T
    rw_T = r_head_w.T
    tw_T = th_head_w.T
    iw_T = inj_head_w.T
    b0T = B_param[..., 0].T
    b1T = B_param[..., 1].T
    c0T = C_param[..., 0].T
    c1T = C_param[..., 1].T
    rb = r_logit_base.reshape(1, M)
    tb = th_atanh_base.reshape(1, M)
    dtb = dt_base.reshape(1, M)
    enc_b2 = enc_b.reshape(1, H)
    conv_b2 = conv_b.reshape(1, H)
    d2 = D.reshape(1, H)

    def fixed(shape):
        return pl.BlockSpec(shape, lambda c: tuple(0 for _ in shape))

    out = pl.pallas_call(
        functools.partial(_dlinoss_kernel, T=T),
        out_shape=jax.ShapeDtypeStruct((L, B, H), jnp.float32),
        grid=(NC,),
        in_specs=[
            pl.BlockSpec((T, NB, H), lambda c: (c, 0, 0)),
            fixed((H, H)), fixed((1, H)), fixed((K, H)), fixed((1, H)),
            fixed((H, M)), fixed((H, M)), fixed((H, M)),
            fixed((H, M)), fixed((H, M)),
            fixed((M, H)), fixed((M, H)),
            fixed((1, M)), fixed((1, M)), fixed((1, M)), fixed((1, H)),
        ],
        out_specs=pl.BlockSpec((T, NB, H), lambda c: (c, 0, 0)),
        scratch_shapes=[
            pltpu.VMEM((K - 1, NB, H), jnp.float32),
            pltpu.VMEM((NB, M), jnp.float32),
            pltpu.VMEM((NB, M), jnp.float32),
            pltpu.VMEM((NB, M), jnp.float32),
            pltpu.VMEM((NB, M), jnp.float32),
            pltpu.VMEM((T * NB, M), jnp.float32),
            pltpu.VMEM((T * NB, M), jnp.float32),
            pltpu.VMEM((T * NB, M), jnp.float32),
            pltpu.VMEM((T * NB, M), jnp.float32),
            pltpu.VMEM((T * NB, M), jnp.float32),
            pltpu.VMEM((T * NB, M), jnp.float32),
        ],
        compiler_params=pltpu.CompilerParams(
            dimension_semantics=("arbitrary",),
            vmem_limit_bytes=100 * 1024 * 1024,
        ),
        name="selective_dlinoss",
    )(x_t, enc_wT, enc_b2, convw, conv_b2, rw_T, tw_T, iw_T,
      b0T, b1T, c0T, c1T, rb, tb, dtb, d2)

    return jnp.transpose(out, (1, 0, 2))


# speedup vs baseline: 85.7230x; 1.4598x over previous
import functools
import math

import jax
import jax.numpy as jnp
from jax.experimental import pallas as pl
from jax.experimental.pallas import tpu as pltpu

H = 256
M = 256
K = 4
NB = 16

_COS_COEF = (
    0.9999999998846172,
    -0.49999999850320187,
    0.04166666347767178,
    -0.0013888862973696803,
    2.4800551328531826e-05,
    -2.753476742453543e-07,
    2.0603329868034754e-09,
    -9.721733557966761e-12,
)


def _cos_poly(theta):
    u = theta * theta
    acc = jnp.full_like(u, _COS_COEF[-1])
    for c in _COS_COEF[-2::-1]:
        acc = acc * u + c
    return acc


def _silu(x):
    return x * jax.nn.sigmoid(x)


def _mm(x, w):
    return jax.lax.dot_general(x, w, (((1,), (0,)), ((), ())),
                               preferred_element_type=jnp.float32)


def _dlinoss_kernel(
    x_hbm,
    enc_wT_ref,
    enc_b_ref,
    convw_ref,
    conv_b_ref,
    rw_T_ref,
    tw_T_ref,
    iw_T_ref,
    b0T_ref,
    b1T_ref,
    c0T_ref,
    c1T_ref,
    rb_ref,
    tb_ref,
    dtb_ref,
    d_ref,
    o_hbm,
    xtld_ref,
    obuf_ref,
    tail_ref,
    zr_ref, zi_ref, xr_ref, xi_ref,
    sv_ref, p_ref, bzr_ref, bzi_ref,
    xsr_ref, xsi_ref,
    in_sem,
    out_sem,
    *, T, NC):
    c = pl.program_id(0)
    TN = T * NB
    slot = jax.lax.rem(c, 2)
    nslot = jax.lax.rem(c + 1, 2)

    def in_copy(chunk, s, b):
        return pltpu.make_async_copy(
            x_hbm.at[b, pl.ds(chunk * T, T), :],
            xtld_ref.at[s, :, b, :],
            in_sem.at[s])

    def out_copy(chunk, s, b):
        return pltpu.make_async_copy(
            obuf_ref.at[s, :, b, :],
            o_hbm.at[b, pl.ds(chunk * T, T), :],
            out_sem.at[s])

    @pl.when(c == 0)
    def _init():
        tail_ref[...] = jnp.zeros_like(tail_ref)
        zr_ref[...] = jnp.zeros_like(zr_ref)
        zi_ref[...] = jnp.zeros_like(zi_ref)
        xr_ref[...] = jnp.zeros_like(xr_ref)
        xi_ref[...] = jnp.zeros_like(xi_ref)
        for b in range(NB):
            in_copy(0, 0, b).start()

    @pl.when(c + 1 < NC)
    def _prefetch():
        for b in range(NB):
            in_copy(c + 1, nslot, b).start()

    for b in range(NB):
        in_copy(c, slot, b).wait()

    x3 = xtld_ref[slot]
    x2 = x3.reshape(TN, H)

    pre2 = _silu(_mm(x2, enc_wT_ref[...]) + enc_b_ref[...])
    pre3 = pre2.reshape(T, NB, H)

    full = jnp.concatenate([tail_ref[...], pre3], axis=0)
    tail_ref[...] = pre3[T - (K - 1):]
    w = convw_ref[...]
    conv3 = (w[0].reshape(1, 1, H) * full[0:T]
             + w[1].reshape(1, 1, H) * full[1:T + 1]
             + w[2].reshape(1, 1, H) * full[2:T + 2]
             + w[3].reshape(1, 1, H) * pre3)
    feats2 = _silu(conv3.reshape(TN, H) + conv_b_ref[...])

    r = jax.nn.sigmoid(rb_ref[...] + _mm(feats2, rw_T_ref[...]))
    theta = math.pi * jnp.tanh(tb_ref[...] + _mm(feats2, tw_T_ref[...]))
    gate = jax.nn.sigmoid(_mm(feats2, iw_T_ref[...]))

    dtc = jnp.maximum(jax.nn.sigmoid(dtb_ref[...]), 1e-6)
    neg_inv_dtc = -1.0 / dtc
    r2 = jnp.maximum(r * r, 1e-8)
    q = jnp.maximum(r2 - 2.0 * r * _cos_poly(theta) + 1.0, 0.0)
    gd = (dtc * r2) * gate

    bur = _mm(x2, b0T_ref[...])
    bui = _mm(x2, b1T_ref[...])

    sv_ref[...] = r2
    p_ref[...] = q * neg_inv_dtc
    bzr_ref[...] = gd * bur
    bzi_ref[...] = gd * bui

    @pl.when(c >= 2)
    def _wait_prev_out():
        for b in range(NB):
            out_copy(c - 2, slot, b).wait()

    def step(t, carry):
        zr, zi, xr, xi = carry
        b = pl.multiple_of(t * NB, NB)
        sv = sv_ref[pl.ds(b, NB), :]
        pv = p_ref[pl.ds(b, NB), :]
        zr = sv * zr + pv * xr + bzr_ref[pl.ds(b, NB), :]
        zi = sv * zi + pv * xi + bzi_ref[pl.ds(b, NB), :]
        xr = xr + dtc * zr
        xi = xi + dtc * zi
        xsr_ref[pl.ds(b, NB), :] = xr
        xsi_ref[pl.ds(b, NB), :] = xi
        return (zr, zi, xr, xi)

    init = (zr_ref[...], zi_ref[...], xr_ref[...], xi_ref[...])
    zr, zi, xr, xi = jax.lax.fori_loop(0, T, step, init)
    zr_ref[...] = zr
    zi_ref[...] = zi
    xr_ref[...] = xr
    xi_ref[...] = xi

    proj = _mm(xsr_ref[...], c0T_ref[...]) - _mm(xsi_ref[...], c1T_ref[...])
    obuf_ref[slot] = proj.reshape(T, NB, H) + d_ref[...].reshape(1, 1, H) * x3

    for b in range(NB):
        out_copy(c, slot, b).start()

    @pl.when(c == NC - 1)
    def _drain():
        @pl.when(c >= 1)
        def _():
            for b in range(NB):
                out_copy(c - 1, nslot, b).wait()
        for b in range(NB):
            out_copy(c, slot, b).wait()


def kernel(inputs, B_param, C_param, D, enc_w, enc_b, conv_w, conv_b,
           r_logit_base, th_atanh_base, r_head_w, th_head_w, dt_base,
           inj_head_w):
    B, L, _ = inputs.shape
    T = 128
    NC = L // T

    enc_wT = enc_w.T
    convw = conv_w[:, 0, :].T
    rw_T = r_head_w.T
    tw_T = th_head_w.T
    iw_T = inj_head_w.T
    b0T = B_param[..., 0].T
    b1T = B_param[..., 1].T
    c0T = C_param[..., 0].T
    c1T = C_param[..., 1].T
    rb = r_logit_base.reshape(1, M)
    tb = th_atanh_base.reshape(1, M)
    dtb = dt_base.reshape(1, M)
    enc_b2 = enc_b.reshape(1, H)
    conv_b2 = conv_b.reshape(1, H)
    d2 = D.reshape(1, H)

    def fixed(shape):
        return pl.BlockSpec(shape, lambda c: tuple(0 for _ in shape))

    out = pl.pallas_call(
        functools.partial(_dlinoss_kernel, T=T, NC=NC),
        out_shape=jax.ShapeDtypeStruct((B, L, H), jnp.float32),
        grid=(NC,),
        in_specs=[
            pl.BlockSpec(memory_space=pl.ANY),
            fixed((H, H)), fixed((1, H)), fixed((K, H)), fixed((1, H)),
            fixed((H, M)), fixed((H, M)), fixed((H, M)),
            fixed((H, M)), fixed((H, M)),
            fixed((M, H)), fixed((M, H)),
            fixed((1, M)), fixed((1, M)), fixed((1, M)), fixed((1, H)),
        ],
        out_specs=pl.BlockSpec(memory_space=pl.ANY),
        scratch_shapes=[
            pltpu.VMEM((2, T, NB, H), jnp.float32),
            pltpu.VMEM((2, T, NB, H), jnp.float32),
            pltpu.VMEM((K - 1, NB, H), jnp.float32),
            pltpu.VMEM((NB, M), jnp.float32),
            pltpu.VMEM((NB, M), jnp.float32),
            pltpu.VMEM((NB, M), jnp.float32),
            pltpu.VMEM((NB, M), jnp.float32),
            pltpu.VMEM((T * NB, M), jnp.float32),
            pltpu.VMEM((T * NB, M), jnp.float32),
            pltpu.VMEM((T * NB, M), jnp.float32),
            pltpu.VMEM((T * NB, M), jnp.float32),
            pltpu.VMEM((T * NB, M), jnp.float32),
            pltpu.VMEM((T * NB, M), jnp.float32),
            pltpu.SemaphoreType.DMA((2,)),
            pltpu.SemaphoreType.DMA((2,)),
        ],
        compiler_params=pltpu.CompilerParams(
            dimension_semantics=("arbitrary",),
            vmem_limit_bytes=100 * 1024 * 1024,
        ),
        name="selective_dlinoss",
    )(inputs, enc_wT, enc_b2, convw, conv_b2, rw_T, tw_T, iw_T,
      b0T, b1T, c0T, c1T, rb, tb, dtb, d2)

    return out


# speedup vs baseline: 95.0555x; 1.1089x over previous
import functools
import math

import jax
import jax.numpy as jnp
from jax.experimental import pallas as pl
from jax.experimental.pallas import tpu as pltpu

H = 256
M = 256
K = 4
NB = 16

_COS_COEF = (
    0.9999999889445765,
    -4.9348011166440395,
    4.058694745521683,
    -1.3351580223048074,
    0.23502902262478848,
    -0.025358285754444,
    0.0015936782135993002,
)


def _cos_pi_tanh(w):
    u = w * w
    acc = jnp.full_like(u, _COS_COEF[-1])
    for c in _COS_COEF[-2::-1]:
        acc = acc * u + c
    return acc


def _sigmoid(x):
    return 0.5 + 0.5 * jnp.tanh(0.5 * x)


def _silu(x):
    h = 0.5 * x
    return h + h * jnp.tanh(h)


def _mm(x, w):
    return jax.lax.dot_general(x, w, (((1,), (0,)), ((), ())),
                               preferred_element_type=jnp.float32)


def _dlinoss_kernel(
    x_hbm,
    enc_wT_ref,
    enc_b_ref,
    convw_ref,
    conv_b_ref,
    rw_T_ref,
    tw_T_ref,
    iw_T_ref,
    b0T_ref,
    b1T_ref,
    c0T_ref,
    c1T_ref,
    rb_ref,
    tb_ref,
    dtb_ref,
    d_ref,
    o_hbm,
    xtld_ref,
    obuf_ref,
    tail_ref,
    zr_ref, zi_ref, xr_ref, xi_ref,
    sv_ref, p_ref, bzr_ref, bzi_ref,
    xsr_ref, xsi_ref,
    in_sem,
    out_sem,
    *, T, NC):
    c = pl.program_id(0)
    TN = T * NB
    slot = jax.lax.rem(c, 2)
    nslot = jax.lax.rem(c + 1, 2)

    def in_copy(chunk, s, b):
        return pltpu.make_async_copy(
            x_hbm.at[b, pl.ds(chunk * T, T), :],
            xtld_ref.at[s, :, b, :],
            in_sem.at[s])

    def out_copy(chunk, s, b):
        return pltpu.make_async_copy(
            obuf_ref.at[s, :, b, :],
            o_hbm.at[b, pl.ds(chunk * T, T), :],
            out_sem.at[s])

    @pl.when(c == 0)
    def _init():
        tail_ref[...] = jnp.zeros_like(tail_ref)
        zr_ref[...] = jnp.zeros_like(zr_ref)
        zi_ref[...] = jnp.zeros_like(zi_ref)
        xr_ref[...] = jnp.zeros_like(xr_ref)
        xi_ref[...] = jnp.zeros_like(xi_ref)
        for b in range(NB):
            in_copy(0, 0, b).start()

    @pl.when(c + 1 < NC)
    def _prefetch():
        for b in range(NB):
            in_copy(c + 1, nslot, b).start()

    for b in range(NB):
        in_copy(c, slot, b).wait()

    x3 = xtld_ref[slot]
    x2 = x3.reshape(TN, H)

    pre2 = _silu(_mm(x2, enc_wT_ref[...]) + enc_b_ref[...])
    pre3 = pre2.reshape(T, NB, H)

    full = jnp.concatenate([tail_ref[...], pre3], axis=0)
    tail_ref[...] = pre3[T - (K - 1):]
    w = convw_ref[...]
    conv3 = (w[0].reshape(1, 1, H) * full[0:T]
             + w[1].reshape(1, 1, H) * full[1:T + 1]
             + w[2].reshape(1, 1, H) * full[2:T + 2]
             + w[3].reshape(1, 1, H) * pre3)
    feats2 = _silu(conv3.reshape(TN, H) + conv_b_ref[...])

    r = _sigmoid(rb_ref[...] + _mm(feats2, rw_T_ref[...]))
    w_th = jnp.tanh(tb_ref[...] + _mm(feats2, tw_T_ref[...]))
    gate = _sigmoid(_mm(feats2, iw_T_ref[...]))

    dtc = jnp.maximum(jax.nn.sigmoid(dtb_ref[...]), 1e-6)
    neg_inv_dtc = -1.0 / dtc
    r2 = jnp.maximum(r * r, 1e-8)
    q = jnp.maximum(r2 - 2.0 * r * _cos_pi_tanh(w_th) + 1.0, 0.0)
    gd = (dtc * r2) * gate

    bur = _mm(x2, b0T_ref[...])
    bui = _mm(x2, b1T_ref[...])

    sv_ref[...] = r2
    p_ref[...] = q * neg_inv_dtc
    bzr_ref[...] = gd * bur
    bzi_ref[...] = gd * bui

    @pl.when(c >= 2)
    def _wait_prev_out():
        for b in range(NB):
            out_copy(c - 2, slot, b).wait()

    def step(t, carry):
        zr, zi, xr, xi = carry
        b = pl.multiple_of(t * NB, NB)
        sv = sv_ref[pl.ds(b, NB), :]
        pv = p_ref[pl.ds(b, NB), :]
        zr = sv * zr + pv * xr + bzr_ref[pl.ds(b, NB), :]
        zi = sv * zi + pv * xi + bzi_ref[pl.ds(b, NB), :]
        xr = xr + dtc * zr
        xi = xi + dtc * zi
        xsr_ref[pl.ds(b, NB), :] = xr
        xsi_ref[pl.ds(b, NB), :] = xi
        return (zr, zi, xr, xi)

    init = (zr_ref[...], zi_ref[...], xr_ref[...], xi_ref[...])
    zr, zi, xr, xi = jax.lax.fori_loop(0, T, step, init, unroll=2)
    zr_ref[...] = zr
    zi_ref[...] = zi
    xr_ref[...] = xr
    xi_ref[...] = xi

    proj = _mm(xsr_ref[...], c0T_ref[...]) - _mm(xsi_ref[...], c1T_ref[...])
    obuf_ref[slot] = proj.reshape(T, NB, H) + d_ref[...].reshape(1, 1, H) * x3

    for b in range(NB):
        out_copy(c, slot, b).start()

    @pl.when(c == NC - 1)
    def _drain():
        @pl.when(c >= 1)
        def _():
            for b in range(NB):
                out_copy(c - 1, nslot, b).wait()
        for b in range(NB):
            out_copy(c, slot, b).wait()


def kernel(inputs, B_param, C_param, D, enc_w, enc_b, conv_w, conv_b,
           r_logit_base, th_atanh_base, r_head_w, th_head_w, dt_base,
           inj_head_w):
    B, L, _ = inputs.shape
    T = 128
    NC = L // T

    enc_wT = enc_w.T
    convw = conv_w[:, 0, :].T
    rw_T = r_head_w.T
    tw_T = th_head_w.T
    iw_T = inj_head_w.T
    b0T = B_param[..., 0].T
    b1T = B_param[..., 1].T
    c0T = C_param[..., 0].T
    c1T = C_param[..., 1].T
    rb = r_logit_base.reshape(1, M)
    tb = th_atanh_base.reshape(1, M)
    dtb = dt_base.reshape(1, M)
    enc_b2 = enc_b.reshape(1, H)
    conv_b2 = conv_b.reshape(1, H)
    d2 = D.reshape(1, H)

    def fixed(shape):
        return pl.BlockSpec(shape, lambda c: tuple(0 for _ in shape))

    out = pl.pallas_call(
        functools.partial(_dlinoss_kernel, T=T, NC=NC),
        out_shape=jax.ShapeDtypeStruct((B, L, H), jnp.float32),
        grid=(NC,),
        in_specs=[
            pl.BlockSpec(memory_space=pl.ANY),
            fixed((H, H)), fixed((1, H)), fixed((K, H)), fixed((1, H)),
            fixed((H, M)), fixed((H, M)), fixed((H, M)),
            fixed((H, M)), fixed((H, M)),
            fixed((M, H)), fixed((M, H)),
            fixed((1, M)), fixed((1, M)), fixed((1, M)), fixed((1, H)),
        ],
        out_specs=pl.BlockSpec(memory_space=pl.ANY),
        scratch_shapes=[
            pltpu.VMEM((2, T, NB, H), jnp.float32),
            pltpu.VMEM((2, T, NB, H), jnp.float32),
            pltpu.VMEM((K - 1, NB, H), jnp.float32),
            pltpu.VMEM((NB, M), jnp.float32),
            pltpu.VMEM((NB, M), jnp.float32),
            pltpu.VMEM((NB, M), jnp.float32),
            pltpu.VMEM((NB, M), jnp.float32),
            pltpu.VMEM((T * NB, M), jnp.float32),
            pltpu.VMEM((T * NB, M), jnp.float32),
            pltpu.VMEM((T * NB, M), jnp.float32),
            pltpu.VMEM((T * NB, M), jnp.float32),
            pltpu.VMEM((T * NB, M), jnp.float32),
            pltpu.VMEM((T * NB, M), jnp.float32),
            pltpu.SemaphoreType.DMA((2,)),
            pltpu.SemaphoreType.DMA((2,)),
        ],
        compiler_params=pltpu.CompilerParams(
            dimension_semantics=("arbitrary",),
            vmem_limit_bytes=100 * 1024 * 1024,
        ),
        name="selective_dlinoss",
    )(inputs, enc_wT, enc_b2, convw, conv_b2, rw_T, tw_T, iw_T,
      b0T, b1T, c0T, c1T, rb, tb, dtb, d2)

    return out
